# baseline (speedup 1.0000x reference)
import jax
import jax.numpy as jnp
from jax import lax
from jax.experimental import pallas as pl
from jax.experimental.pallas import tpu as pltpu
from jax.experimental.pallas import tpu_sc as plsc

_C, _H, _W = 3, 224, 224
_N = 256
_CHUNKS = _C * _H
_NWORKERS = 32
_PER_W = _CHUNKS // _NWORKERS


def _sc_body(t_hbm, o_hbm, buf, insem, outsem):
    wid = lax.axis_index("s") * 2 + lax.axis_index("c")
    base = wid * _PER_W

    def chunk_idx(step):
        idx = base + step
        return idx // _H, idx % _H

    def start_in(step, b):
        c, h = chunk_idx(step)
        pltpu.make_async_copy(t_hbm.at[c, h], buf.at[b], insem.at[b]).start()

    def wait_in(step, b):
        c, h = chunk_idx(step)
        pltpu.make_async_copy(t_hbm.at[c, h], buf.at[b], insem.at[b]).wait()

    def start_out(step, b):
        c, h = chunk_idx(step)
        pltpu.make_async_copy(buf.at[b], o_hbm.at[c, h], outsem.at[b]).start()

    def wait_out(step, b):
        c, h = chunk_idx(step)
        pltpu.make_async_copy(buf.at[b], o_hbm.at[c, h], outsem.at[b]).wait()

    def compute(b):
        bufb = buf.at[b]

        def row(r, _):
            for l in range(_N // 16):
                v = bufb[r, pl.ds(16 * l, 16)]
                y = jnp.minimum(jnp.maximum(v * 255.0, 0.0), 255.0)
                k = (y + 0.5).astype(jnp.int32)
                bufb[r, pl.ds(16 * l, 16)] = k.astype(jnp.float32) * (1.0 / 255.0)
            return _

        lax.fori_loop(0, _W, row, None)

    start_in(0, 0)
    for step in range(_PER_W):
        b = step % 2
        wait_in(step, b)
        if step + 1 < _PER_W:
            if step >= 1:
                wait_out(step - 1, 1 - b)
            start_in(step + 1, 1 - b)
        compute(b)
        start_out(step, b)
    wait_out(_PER_W - 2, _PER_W % 2)
    wait_out(_PER_W - 1, (_PER_W - 1) % 2)


def kernel(watermark_samples, response):
    t = jnp.transpose(watermark_samples, (1, 2, 3, 0))
    sc_call = pl.kernel(
        _sc_body,
        out_type=jax.ShapeDtypeStruct((_C, _H, _W, _N), jnp.float32),
        mesh=plsc.VectorSubcoreMesh(core_axis_name="c", subcore_axis_name="s"),
        scratch_types=[
            pltpu.VMEM((2, _W, _N), jnp.float32),
            pltpu.SemaphoreType.DMA((2,)),
            pltpu.SemaphoreType.DMA((2,)),
        ],
    )
    out = sc_call(t)
    return (jnp.transpose(out, (3, 0, 1, 2)), response)


# speedup vs baseline: 2.1394x; 2.1394x over previous
import jax
import jax.numpy as jnp
from jax.experimental import pallas as pl

_BLOCK_H = 28


def _discretize_body(w_ref, o_ref):
    x = jnp.clip(w_ref[...], 0.0, 1.0)
    o_ref[...] = jnp.round(x * 255.0) / 255.0


def kernel(watermark_samples, response):
    n, c, h, w = watermark_samples.shape
    t = jnp.transpose(watermark_samples, (1, 2, 3, 0))
    out = pl.pallas_call(
        _discretize_body,
        grid=(c, h // _BLOCK_H),
        in_specs=[pl.BlockSpec((1, _BLOCK_H, w, n), lambda i, j: (i, j, 0, 0))],
        out_specs=pl.BlockSpec((1, _BLOCK_H, w, n), lambda i, j: (i, j, 0, 0)),
        out_shape=jax.ShapeDtypeStruct((c, h, w, n), jnp.float32),
    )(t)
    return (jnp.transpose(out, (3, 0, 1, 2)), response)


# speedup vs baseline: 2.1751x; 1.0167x over previous
import jax
import jax.numpy as jnp
from jax.experimental import pallas as pl

_BLOCK_H = 56


def _discretize_body(w_ref, o_ref):
    x = jnp.clip(w_ref[...], 0.0, 1.0)
    o_ref[...] = jnp.round(x * 255.0) / 255.0


def kernel(watermark_samples, response):
    n, c, h, w = watermark_samples.shape
    t = jnp.transpose(watermark_samples, (1, 2, 3, 0))
    out = pl.pallas_call(
        _discretize_body,
        grid=(c, h // _BLOCK_H),
        in_specs=[pl.BlockSpec((1, _BLOCK_H, w, n), lambda i, j: (i, j, 0, 0))],
        out_specs=pl.BlockSpec((1, _BLOCK_H, w, n), lambda i, j: (i, j, 0, 0)),
        out_shape=jax.ShapeDtypeStruct((c, h, w, n), jnp.float32),
    )(t)
    return (jnp.transpose(out, (3, 0, 1, 2)), response)


# speedup vs baseline: 2.1812x; 1.0028x over previous
import jax
import jax.numpy as jnp
from jax.experimental import pallas as pl
from jax.experimental.pallas import tpu as pltpu

_CHUNK_H = 28
_NBUF = 3


def _body(t_hbm, o_hbm, inb, outb, insem, outsem):
    c, h = t_hbm.shape[0], t_hbm.shape[1]
    steps_per_c = h // _CHUNK_H
    steps = c * steps_per_c

    def src(i):
        return t_hbm.at[i // steps_per_c, pl.ds((i % steps_per_c) * _CHUNK_H, _CHUNK_H)]

    def dst(i):
        return o_hbm.at[i // steps_per_c, pl.ds((i % steps_per_c) * _CHUNK_H, _CHUNK_H)]

    def start_in(i):
        pltpu.make_async_copy(src(i), inb.at[i % _NBUF], insem.at[i % _NBUF]).start()

    def wait_in(i):
        pltpu.make_async_copy(src(i), inb.at[i % _NBUF], insem.at[i % _NBUF]).wait()

    def start_out(i):
        pltpu.make_async_copy(outb.at[i % _NBUF], dst(i), outsem.at[i % _NBUF]).start()

    def wait_out(i):
        pltpu.make_async_copy(outb.at[i % _NBUF], dst(i), outsem.at[i % _NBUF]).wait()

    for i in range(_NBUF):
        start_in(i)
    for i in range(steps):
        b = i % _NBUF
        wait_in(i)
        if i >= _NBUF:
            wait_out(i - _NBUF)
        x = jnp.clip(inb[b], 0.0, 1.0)
        outb[b] = jnp.round(x * 255.0) / 255.0
        start_out(i)
        if i + _NBUF < steps:
            start_in(i + _NBUF)
    for i in range(steps - _NBUF, steps):
        wait_out(i)


def kernel(watermark_samples, response):
    n, c, h, w = watermark_samples.shape
    t = jnp.transpose(watermark_samples, (1, 2, 3, 0))
    out = pl.pallas_call(
        _body,
        in_specs=[pl.BlockSpec(memory_space=pltpu.HBM)],
        out_specs=pl.BlockSpec(memory_space=pltpu.HBM),
        out_shape=jax.ShapeDtypeStruct((c, h, w, n), jnp.float32),
        scratch_shapes=[
            pltpu.VMEM((_NBUF, _CHUNK_H, w, n), jnp.float32),
            pltpu.VMEM((_NBUF, _CHUNK_H, w, n), jnp.float32),
            pltpu.SemaphoreType.DMA((_NBUF,)),
            pltpu.SemaphoreType.DMA((_NBUF,)),
        ],
    )(t)
    return (jnp.transpose(out, (3, 0, 1, 2)), response)
